# baseline (speedup 1.0000x reference)
import functools

import jax
import jax.numpy as jnp
from jax import lax
from jax.experimental import pallas as pl
from jax.experimental.pallas import tpu as pltpu
from jax.experimental.pallas import tpu_sc as plsc

_SHIFT = 8192
_NUM_WORKERS = 32
_SEGS = 120


def _build_shift_kernel(nblocks, in_tb, out_tb, channels, lanes, blocks_per_worker):
    mesh = plsc.VectorSubcoreMesh(core_axis_name="c", subcore_axis_name="s")
    segs = _SEGS
    assert out_tb % segs == 0
    nchunks = out_tb // segs
    nsteps = blocks_per_worker * nchunks

    @functools.partial(
        pl.kernel,
        mesh=mesh,
        out_type=jax.ShapeDtypeStruct((nblocks, out_tb, channels, lanes), jnp.float32),
        scratch_types=[
            pltpu.VMEM((nblocks + 16,), jnp.int32),
            pltpu.VMEM((segs + 1, channels, lanes), jnp.float32),
            pltpu.VMEM((segs + 1, channels, lanes), jnp.float32),
            pltpu.VMEM((segs, channels, lanes), jnp.float32),
            pltpu.VMEM((segs, channels, lanes), jnp.float32),
            pltpu.SemaphoreType.DMA,
            pltpu.SemaphoreType.DMA,
            pltpu.SemaphoreType.DMA,
            pltpu.SemaphoreType.DMA,
        ],
        compiler_params=pltpu.CompilerParams(
            use_tc_tiling_on_sc=False, needs_layout_passes=False
        ),
    )
    def shift_kernel(
        x_hbm, offs_hbm, out_hbm,
        offs_v, bin0, bin1, bout0, bout1, si0, si1, so0, so1,
    ):
        wid = lax.axis_index("s") * 2 + lax.axis_index("c")
        base_blk = wid * blocks_per_worker
        pltpu.sync_copy(offs_hbm, offs_v)
        offs_vec = offs_v[pl.ds(base_blk, 16)]
        lane = jax.lax.iota(jnp.int32, 16)
        qs, rems = [], []
        for b in range(blocks_per_worker):
            off = offs_vec[b]
            q = off // lanes
            qs.append(q)
            rems.append(off - q * lanes)
        bins = [bin0, bin1]
        bouts = [bout0, bout1]
        isems = [si0, si1]
        osems = [so0, so1]

        def in_copy(g):
            b, c = divmod(g, nchunks)
            return pltpu.async_copy(
                x_hbm.at[base_blk + b, pl.ds(qs[b] + c * segs, segs + 1), :, :],
                bins[g % 2],
                isems[g % 2],
            )

        def out_copy(g):
            b, c = divmod(g, nchunks)
            return pltpu.async_copy(
                bouts[g % 2],
                out_hbm.at[base_blk + b, pl.ds(c * segs, segs), :, :],
                osems[g % 2],
            )

        in_handles = {0: in_copy(0)}
        out_handles = {}
        for g in range(nsteps):
            if g + 1 < nsteps:
                in_handles[g + 1] = in_copy(g + 1)
            in_handles[g].wait()
            if g >= 2:
                out_handles[(g - 2, 0)].wait()
                out_handles[(g - 2, 1)].wait()
            b, c = divmod(g, nchunks)
            rem = rems[b]
            bi = bins[g % 2]
            bo = bouts[g % 2]
            carries, wmods = [], []
            lane_bits = lanes.bit_length() - 1
            for jg in range(lanes // 16):
                w = rem + jg * 16 + lane
                carries.append(lax.shift_right_logical(w, lane_bits))
                wmods.append(jnp.bitwise_and(w, lanes - 1))
            cvecs = [jnp.full((16,), ch, jnp.int32) for ch in range(channels)]

            half = segs // 2
            for h in range(2):
                @plsc.parallel_loop(h * half, (h + 1) * half, unroll=4)
                def _seg(s):
                    bis = bi.at[pl.ds(s, 2)]
                    for ch in range(channels):
                        for jg in range(lanes // 16):
                            val = plsc.load_gather(
                                bis, [carries[jg], cvecs[ch], wmods[jg]]
                            )
                            bo[s, ch, pl.ds(jg * 16, 16)] = val

                out_handles[(g, h)] = pltpu.async_copy(
                    bo.at[pl.ds(h * half, half)],
                    out_hbm.at[
                        base_blk + b, pl.ds(c * segs + h * half, half), :, :
                    ],
                    osems[g % 2],
                )
        for g in (nsteps - 2, nsteps - 1):
            for h in range(2):
                out_handles[(g, h)].wait()

    return shift_kernel


def kernel(wav):
    batch, sources, channels, time = wav.shape
    length = time - _SHIFT
    lanes = 128
    in_tb = time // lanes
    out_tb = length // lanes
    nblocks = batch * sources

    def _make_offs():
        offs_key = jax.random.key(42)
        offsets = jax.random.randint(offs_key, (batch, sources, 1, 1), 0, _SHIFT)
        o = offsets.reshape(nblocks).astype(jnp.int32)
        return jnp.pad(o, (0, 16))

    try:
        import numpy as np

        _cpu = jax.local_devices(backend="cpu")[0]
        with jax.ensure_compile_time_eval(), jax.default_device(_cpu):
            offs = jnp.asarray(np.asarray(_make_offs()))
    except Exception:
        offs = _make_offs()

    blocks_per_worker = nblocks // _NUM_WORKERS
    x = wav.reshape(batch, sources, channels, in_tb, lanes)
    x = x.transpose(0, 1, 3, 2, 4).reshape(nblocks, in_tb, channels, lanes)
    out = _build_shift_kernel(
        nblocks, in_tb, out_tb, channels, lanes, blocks_per_worker
    )(x, offs)
    out = out.reshape(batch, sources, out_tb, channels, lanes)
    out = out.transpose(0, 1, 3, 2, 4).reshape(batch, sources, channels, length)
    return out


# speedup vs baseline: 1.0532x; 1.0532x over previous
import functools

import jax
import jax.numpy as jnp
from jax import lax
from jax.experimental import pallas as pl
from jax.experimental.pallas import tpu as pltpu
from jax.experimental.pallas import tpu_sc as plsc

_SHIFT = 8192
_NUM_WORKERS = 32
_SEGS = 120


def _build_shift_kernel(nblocks, in_tb, out_tb, channels, lanes, blocks_per_worker):
    mesh = plsc.VectorSubcoreMesh(core_axis_name="c", subcore_axis_name="s")
    segs = _SEGS
    assert out_tb % segs == 0
    nchunks = out_tb // segs
    nsteps = blocks_per_worker * nchunks

    @functools.partial(
        pl.kernel,
        mesh=mesh,
        out_type=jax.ShapeDtypeStruct((nblocks, out_tb, channels, lanes), jnp.float32),
        scratch_types=[
            pltpu.VMEM((nblocks + 16,), jnp.int32),
            pltpu.VMEM((segs + 1, channels, lanes), jnp.float32),
            pltpu.VMEM((segs + 1, channels, lanes), jnp.float32),
            pltpu.VMEM((segs, channels, lanes), jnp.float32),
            pltpu.VMEM((segs, channels, lanes), jnp.float32),
            pltpu.SemaphoreType.DMA,
            pltpu.SemaphoreType.DMA,
            pltpu.SemaphoreType.DMA,
            pltpu.SemaphoreType.DMA,
        ],
        compiler_params=pltpu.CompilerParams(
            use_tc_tiling_on_sc=False, needs_layout_passes=False
        ),
    )
    def shift_kernel(
        x_hbm, offs_hbm, out_hbm,
        offs_v, bin0, bin1, bout0, bout1, si0, si1, so0, so1,
    ):
        wid = lax.axis_index("s") * 2 + lax.axis_index("c")
        base_blk = wid * blocks_per_worker
        pltpu.sync_copy(offs_hbm, offs_v)
        offs_vec = offs_v[pl.ds(base_blk, 16)]
        lane = jax.lax.iota(jnp.int32, 16)
        qs, rems = [], []
        for b in range(blocks_per_worker):
            off = offs_vec[b]
            q = off // lanes
            qs.append(q)
            rems.append(off - q * lanes)
        bins = [bin0, bin1]
        bouts = [bout0, bout1]
        isems = [si0, si1]
        osems = [so0, so1]

        def in_copy(g):
            b, c = divmod(g, nchunks)
            return pltpu.async_copy(
                x_hbm.at[base_blk + b, pl.ds(qs[b] + c * segs, segs + 1), :, :],
                bins[g % 2],
                isems[g % 2],
            )

        def out_copy(g):
            b, c = divmod(g, nchunks)
            return pltpu.async_copy(
                bouts[g % 2],
                out_hbm.at[base_blk + b, pl.ds(c * segs, segs), :, :],
                osems[g % 2],
            )

        in_handles = {0: in_copy(0)}
        out_handles = {}
        for g in range(nsteps):
            if g + 1 < nsteps:
                in_handles[g + 1] = in_copy(g + 1)
            in_handles[g].wait()
            if g >= 2:
                out_handles[g - 2].wait()
            b, c = divmod(g, nchunks)
            rem = rems[b]
            bi = bins[g % 2]
            bo = bouts[g % 2]
            carries, wmods = [], []
            lane_bits = lanes.bit_length() - 1
            for jg in range(lanes // 16):
                w = rem + jg * 16 + lane
                carries.append(lax.shift_right_logical(w, lane_bits))
                wmods.append(jnp.bitwise_and(w, lanes - 1))
            cvecs = [jnp.full((16,), ch, jnp.int32) for ch in range(channels)]

            @plsc.parallel_loop(0, segs, unroll=4)
            def _seg(s):
                bis = bi.at[pl.ds(s, 2)]
                for ch in range(channels):
                    for jg in range(lanes // 16):
                        val = plsc.load_gather(
                            bis, [carries[jg], cvecs[ch], wmods[jg]]
                        )
                        bo[s, ch, pl.ds(jg * 16, 16)] = val

            out_handles[g] = out_copy(g)
        out_handles[nsteps - 2].wait()
        out_handles[nsteps - 1].wait()

    return shift_kernel


def kernel(wav):
    batch, sources, channels, time = wav.shape
    length = time - _SHIFT
    lanes = 128
    in_tb = time // lanes
    out_tb = length // lanes
    nblocks = batch * sources

    def _make_offs():
        offs_key = jax.random.key(42)
        offsets = jax.random.randint(offs_key, (batch, sources, 1, 1), 0, _SHIFT)
        o = offsets.reshape(nblocks).astype(jnp.int32)
        return jnp.pad(o, (0, 16))

    try:
        import numpy as np

        _cpu = jax.local_devices(backend="cpu")[0]
        with jax.ensure_compile_time_eval(), jax.default_device(_cpu):
            offs = jnp.asarray(np.asarray(_make_offs()))
    except Exception:
        offs = _make_offs()

    blocks_per_worker = nblocks // _NUM_WORKERS
    x = wav.reshape(batch, sources, channels, in_tb, lanes)
    x = x.transpose(0, 1, 3, 2, 4).reshape(nblocks, in_tb, channels, lanes)
    out = _build_shift_kernel(
        nblocks, in_tb, out_tb, channels, lanes, blocks_per_worker
    )(x, offs)
    out = out.reshape(batch, sources, out_tb, channels, lanes)
    out = out.transpose(0, 1, 3, 2, 4).reshape(batch, sources, channels, length)
    return out
